# baseline (device time: 98684 ns/iter reference)
import jax
import jax.numpy as jnp
from jax import lax
from jax.experimental import pallas as pl
from jax.experimental.pallas import tpu as pltpu

N_DEV = 8


def _gather_counts(counts_pad):

    def body(cnt_ref, all_ref, send_sem, recv_sem):
        me = lax.axis_index("i")

        barrier_sem = pltpu.get_barrier_semaphore()
        for o in range(1, N_DEV):
            pl.semaphore_signal(
                barrier_sem, inc=1,
                device_id=(lax.rem(me + o, N_DEV),),
                device_id_type=pl.DeviceIdType.MESH,
            )
        pl.semaphore_wait(barrier_sem, N_DEV - 1)

        all_ref[pl.ds(me, 1), :] = cnt_ref[:, :]
        for o in range(1, N_DEV):
            tgt = lax.rem(me + o, N_DEV)
            rdma = pltpu.make_async_remote_copy(
                src_ref=cnt_ref,
                dst_ref=all_ref.at[pl.ds(me, 1), :],
                send_sem=send_sem,
                recv_sem=recv_sem,
                device_id=(tgt,),
                device_id_type=pl.DeviceIdType.MESH,
            )
            rdma.start()

        dummy = pltpu.make_async_remote_copy(
            src_ref=cnt_ref,
            dst_ref=all_ref.at[pl.ds(me, 1), :],
            send_sem=send_sem,
            recv_sem=recv_sem,
            device_id=(0,),
            device_id_type=pl.DeviceIdType.MESH,
        )
        for _ in range(N_DEV - 1):
            dummy.wait_recv()
        for _ in range(N_DEV - 1):
            dummy.wait_send()

    return pl.pallas_call(
        body,
        out_shape=jax.ShapeDtypeStruct((N_DEV, 128), jnp.int32),
        in_specs=[pl.BlockSpec(memory_space=pltpu.VMEM)],
        out_specs=pl.BlockSpec(memory_space=pltpu.VMEM),
        scratch_shapes=[pltpu.SemaphoreType.DMA, pltpu.SemaphoreType.DMA],
        compiler_params=pltpu.CompilerParams(collective_id=0),
    )(counts_pad)


def _a2av(x, dst_rank, dst_row, n_remote):
    m, n = x.shape

    def body(x_ref, rank_ref, row_ref, nrem_ref, out_ref, send_sem, recv_sem):
        me = lax.axis_index("i")

        barrier_sem = pltpu.get_barrier_semaphore()
        for o in range(1, N_DEV):
            pl.semaphore_signal(
                barrier_sem, inc=1,
                device_id=(lax.rem(me + o, N_DEV),),
                device_id_type=pl.DeviceIdType.MESH,
            )
        pl.semaphore_wait(barrier_sem, N_DEV - 1)

        def send_one(j, carry):
            d = rank_ref[j]
            r = row_ref[j]

            @pl.when(d == me)
            def _():
                out_ref[pl.ds(r, 1), :] = x_ref[pl.ds(j, 1), :]

            @pl.when(d != me)
            def _():
                rdma = pltpu.make_async_remote_copy(
                    src_ref=x_ref.at[pl.ds(j, 1), :],
                    dst_ref=out_ref.at[pl.ds(r, 1), :],
                    send_sem=send_sem,
                    recv_sem=recv_sem,
                    device_id=(d,),
                    device_id_type=pl.DeviceIdType.MESH,
                )
                rdma.start()

            return carry

        lax.fori_loop(0, m, send_one, 0)

        dummy = pltpu.make_async_remote_copy(
            src_ref=x_ref.at[pl.ds(0, 1), :],
            dst_ref=out_ref.at[pl.ds(0, 1), :],
            send_sem=send_sem,
            recv_sem=recv_sem,
            device_id=(0,),
            device_id_type=pl.DeviceIdType.MESH,
        )
        nrem = nrem_ref[0]

        def wait_r(k, carry):
            dummy.wait_recv()
            return carry

        lax.fori_loop(0, nrem, wait_r, 0)

        def wait_s(k, carry):
            dummy.wait_send()
            return carry

        lax.fori_loop(0, nrem, wait_s, 0)

    return pl.pallas_call(
        body,
        out_shape=jax.ShapeDtypeStruct((m, n), x.dtype),
        in_specs=[
            pl.BlockSpec(memory_space=pltpu.VMEM),
            pl.BlockSpec(memory_space=pltpu.SMEM),
            pl.BlockSpec(memory_space=pltpu.SMEM),
            pl.BlockSpec(memory_space=pltpu.SMEM),
        ],
        out_specs=pl.BlockSpec(memory_space=pltpu.VMEM),
        scratch_shapes=[pltpu.SemaphoreType.DMA, pltpu.SemaphoreType.DMA],
        compiler_params=pltpu.CompilerParams(collective_id=1),
    )(x, dst_rank, dst_row, n_remote)


def kernel(x, dest):
    m, _ = x.shape
    me = lax.axis_index("i")
    dest = dest.astype(jnp.int32)

    onehot = (dest[:, None] == jnp.arange(N_DEV, dtype=jnp.int32)[None, :])
    onehot = onehot.astype(jnp.int32)
    counts = jnp.sum(onehot, axis=0)
    excl = jnp.cumsum(onehot, axis=0) - onehot
    rank_within = jnp.take_along_axis(excl, dest[:, None], axis=1)[:, 0]

    counts_pad = jnp.zeros((1, 128), jnp.int32).at[0, :N_DEV].set(counts)
    allcounts = _gather_counts(counts_pad)[:, :N_DEV]

    col_prefix = jnp.cumsum(allcounts, axis=0) - allcounts
    remote_off = lax.dynamic_slice_in_dim(col_prefix, me, 1, axis=0)[0]

    dst_row = jnp.take(remote_off, dest) + rank_within
    n_remote = (m - jnp.take(counts, me)).astype(jnp.int32)[None]

    return _a2av(x, dest, dst_row, n_remote)


# device time: 87160 ns/iter; 1.1322x vs baseline; 1.1322x over previous
import jax
import jax.numpy as jnp
from jax import lax
from jax.experimental import pallas as pl
from jax.experimental.pallas import tpu as pltpu

N_DEV = 8
LANES = 128


def _entry_barrier(me):
    barrier_sem = pltpu.get_barrier_semaphore()
    for o in range(1, N_DEV):
        pl.semaphore_signal(
            barrier_sem, inc=1,
            device_id=(lax.rem(me + o, N_DEV),),
            device_id_type=pl.DeviceIdType.MESH,
        )
    pl.semaphore_wait(barrier_sem, N_DEV - 1)


def _gather_partial_counts(dest_v):

    def body(dest_ref, all_ref, part_ref, send_sem, recv_sem):
        me = lax.axis_index("i")
        _entry_barrier(me)

        for d in range(N_DEV):
            mask = (dest_ref[:, :] == d).astype(jnp.int32)
            part_ref[0, d, :] = jnp.sum(mask, axis=0)
        all_ref[pl.ds(me, 1), :, :] = part_ref[:, :, :]

        for o in range(1, N_DEV):
            tgt = lax.rem(me + o, N_DEV)
            rdma = pltpu.make_async_remote_copy(
                src_ref=part_ref,
                dst_ref=all_ref.at[pl.ds(me, 1), :, :],
                send_sem=send_sem,
                recv_sem=recv_sem,
                device_id=(tgt,),
                device_id_type=pl.DeviceIdType.MESH,
            )
            rdma.start()

        dummy = pltpu.make_async_remote_copy(
            src_ref=part_ref,
            dst_ref=all_ref.at[pl.ds(me, 1), :, :],
            send_sem=send_sem,
            recv_sem=recv_sem,
            device_id=(0,),
            device_id_type=pl.DeviceIdType.MESH,
        )
        for _ in range(N_DEV - 1):
            dummy.wait_recv()
        for _ in range(N_DEV - 1):
            dummy.wait_send()

    return pl.pallas_call(
        body,
        out_shape=jax.ShapeDtypeStruct((N_DEV, N_DEV, LANES), jnp.int32),
        in_specs=[pl.BlockSpec(memory_space=pltpu.VMEM)],
        out_specs=pl.BlockSpec(memory_space=pltpu.VMEM),
        scratch_shapes=[
            pltpu.VMEM((1, N_DEV, LANES), jnp.int32),
            pltpu.SemaphoreType.DMA,
            pltpu.SemaphoreType.DMA,
        ],
        compiler_params=pltpu.CompilerParams(collective_id=0),
    )(dest_v)


def _a2av(x, dest, allcounts_flat):
    m, n = x.shape

    def body(x_ref, dest_ref, cnt_ref, out_ref,
             roff_ref, run_ref, send_sem, recv_sem):
        me = lax.axis_index("i")
        _entry_barrier(me)

        for d in range(N_DEV):
            acc = jnp.int32(0)
            for src in range(N_DEV):
                c = cnt_ref[src * N_DEV + d]
                acc = acc + jnp.where(src < me, c, 0)
            roff_ref[d] = acc
            run_ref[d] = 0

        def send_one(j, carry):
            d = dest_ref[j]
            rw = run_ref[d]
            run_ref[d] = rw + 1
            r = roff_ref[d] + rw

            @pl.when(d == me)
            def _():
                out_ref[pl.ds(r, 1), :] = x_ref[pl.ds(j, 1), :]

            @pl.when(d != me)
            def _():
                rdma = pltpu.make_async_remote_copy(
                    src_ref=x_ref.at[pl.ds(j, 1), :],
                    dst_ref=out_ref.at[pl.ds(r, 1), :],
                    send_sem=send_sem,
                    recv_sem=recv_sem,
                    device_id=(d,),
                    device_id_type=pl.DeviceIdType.MESH,
                )
                rdma.start()

            return carry

        lax.fori_loop(0, m, send_one, 0)

        nrem = m - cnt_ref[me * N_DEV + me]
        dummy = pltpu.make_async_remote_copy(
            src_ref=x_ref.at[pl.ds(0, 1), :],
            dst_ref=out_ref.at[pl.ds(0, 1), :],
            send_sem=send_sem,
            recv_sem=recv_sem,
            device_id=(0,),
            device_id_type=pl.DeviceIdType.MESH,
        )

        def wait_r(k, carry):
            dummy.wait_recv()
            return carry

        lax.fori_loop(0, nrem, wait_r, 0)

        def wait_s(k, carry):
            dummy.wait_send()
            return carry

        lax.fori_loop(0, nrem, wait_s, 0)

    return pl.pallas_call(
        body,
        out_shape=jax.ShapeDtypeStruct((m, n), x.dtype),
        in_specs=[
            pl.BlockSpec(memory_space=pltpu.VMEM),
            pl.BlockSpec(memory_space=pltpu.SMEM),
            pl.BlockSpec(memory_space=pltpu.SMEM),
        ],
        out_specs=pl.BlockSpec(memory_space=pltpu.VMEM),
        scratch_shapes=[
            pltpu.SMEM((N_DEV,), jnp.int32),
            pltpu.SMEM((N_DEV,), jnp.int32),
            pltpu.SemaphoreType.DMA,
            pltpu.SemaphoreType.DMA,
        ],
        compiler_params=pltpu.CompilerParams(collective_id=1),
    )(x, dest, allcounts_flat)


def kernel(x, dest):
    m, _ = x.shape
    dest = dest.astype(jnp.int32)

    dest_v = dest.reshape(m // LANES, LANES)
    partials = _gather_partial_counts(dest_v)
    allcounts_flat = jnp.sum(partials, axis=-1).reshape(N_DEV * N_DEV)

    return _a2av(x, dest, allcounts_flat)
